# Initial kernel scaffold; baseline (speedup 1.0000x reference)
import jax
import jax.numpy as jnp
from jax.experimental import pallas as pl


def kernel(is_last_layer, x, edge_index, edge_attr, p, p_focal_deg1, p_focal_deg2, p_focal_deg3, p_focal_deg4, nei_p_deg1, nei_p_deg2, nei_p_deg3, nei_p_deg4, nei_edge_attr_deg1, nei_edge_attr_deg2, nei_edge_attr_deg3, nei_edge_attr_deg4, selected_index_deg1, selected_index_deg2, selected_index_deg3, selected_index_deg4, nei_index_deg1, nei_index_deg2, nei_index_deg3, nei_index_deg4, save_score, W1, W2, W3, W4):
    raise NotImplementedError("write your pallas kernel here")


# speedup vs baseline: 1.5727x; 1.5727x over previous
import functools

import jax
import jax.numpy as jnp
from jax import lax
from jax.experimental import pallas as pl
from jax.experimental.pallas import tpu as pltpu
from jax.experimental.pallas import tpu_sc as plsc

N_NODES = 100000
N_FOCAL = 25000
D_FEAT = 128
NK = 8

NW = 32
CHUNK = 3200
PAD_BLK = 25600
NE = NW * CHUNK
NB = 100352
PAD_KEY = N_NODES
ZROWS = 2000
PROWS = 4 * N_NODES + ZROWS
RB = 2000
NRB = N_NODES // RB


def _proj_body(x_ref, w_ref, o_ref):
    i = pl.program_id(0)
    d = jnp.minimum(i // NRB, 3)
    lane = lax.broadcasted_iota(jnp.int32, (1, 16), 1)
    scale = jnp.where(lane < 8, 1.0, 1.0 / (d + 1).astype(jnp.float32))
    wmat = w_ref[0] * scale
    res = lax.dot_general(x_ref[...], wmat, (((1,), (0,)), ((), ())),
                          preferred_element_type=jnp.float32)
    o_ref[...] = jnp.where(i >= 4 * NRB, 0.0, res)


def _project(x, wstack):
    grid = 4 * NRB + 1
    return pl.pallas_call(
        _proj_body,
        grid=(grid,),
        in_specs=[
            pl.BlockSpec((RB, D_FEAT), lambda i: (jnp.where(i >= 4 * NRB, 0, i % NRB), 0)),
            pl.BlockSpec((1, D_FEAT, 16), lambda i: (jnp.minimum(i // NRB, 3), 0, 0)),
        ],
        out_specs=pl.BlockSpec((RB, 16), lambda i: (i, 0)),
        out_shape=jax.ShapeDtypeStruct((PROWS, 16), jnp.float32),
    )(x, wstack)


def _offsets_body(h_ref, t_ref, carry_ref):
    pid = pl.program_id(0)

    @pl.when(pid == 0)
    def _():
        carry_ref[0] = 0

    blk = h_ref[...]

    def shift_down0(a, s):
        return jnp.pad(a, ((s, 0), (0, 0)))[: a.shape[0], :]

    def shift_down1(a, s):
        return jnp.pad(a, ((0, 0), (s, 0)))[:, : a.shape[1]]

    cum0 = blk
    s = 1
    while s < 32:
        cum0 = cum0 + shift_down0(cum0, s)
        s *= 2
    wexcl = cum0 - blk

    total = jnp.sum(blk, axis=0, keepdims=True)
    cum1 = total
    s = 1
    while s < 2048:
        cum1 = cum1 + shift_down1(cum1, s)
        s *= 2
    carry = carry_ref[0]
    excl_bins = cum1 - total + carry
    t_ref[...] = wexcl + excl_bins
    carry_ref[0] = carry + jnp.sum(total)


def _offsets(hgrid):
    nblk = NB // 2048
    return pl.pallas_call(
        _offsets_body,
        grid=(nblk,),
        in_specs=[pl.BlockSpec((NW, 2048), lambda i: (0, i))],
        out_specs=pl.BlockSpec((NW, 2048), lambda i: (0, i)),
        out_shape=jax.ShapeDtypeStruct((NW, NB), jnp.int32),
        scratch_shapes=[pltpu.SMEM((1,), jnp.int32)],
    )(hgrid)


_MESH = plsc.VectorSubcoreMesh(core_axis_name="c", subcore_axis_name="s")


def _wid():
    return lax.axis_index("s") * 2 + lax.axis_index("c")


@functools.partial(
    pl.kernel,
    mesh=_MESH,
    compiler_params=pltpu.CompilerParams(needs_layout_passes=False),
    out_type=(
        jax.ShapeDtypeStruct((NW * NB,), jnp.int32),
        jax.ShapeDtypeStruct((NE,), jnp.int32),
    ),
    scratch_types=[
        pltpu.VMEM((NB,), jnp.int32),
        pltpu.VMEM((CHUNK,), jnp.int32),
        pltpu.VMEM((CHUNK,), jnp.int32),
        pltpu.VMEM((48,), jnp.int32),
    ],
)
def _k1(keys_hbm, h_hbm, rank_hbm, hist, keysb, rankb, shf):
    w = _wid()
    pltpu.sync_copy(keys_hbm.at[pl.ds(w * CHUNK, CHUNK)], keysb)

    zero16 = jnp.zeros((16,), jnp.int32)

    def zbody(j, c):
        hist[pl.ds(j * 16, 16)] = zero16
        return c

    lax.fori_loop(0, NB // 16, zbody, 0)

    neg16 = jnp.full((16,), -1, jnp.int32)
    shf[pl.ds(0, 16)] = neg16
    shf[pl.ds(16, 16)] = neg16
    shf[pl.ds(32, 16)] = neg16

    def body(i, c):
        off = i * 16
        kv = keysb[pl.ds(off, 16)]
        shf[pl.ds(15, 16)] = kv
        within = jnp.zeros((16,), jnp.int32)
        after = jnp.zeros((16,), jnp.int32)
        for k in range(1, 16):
            lv = shf[pl.ds(15 - k, 16)]
            within = within + jnp.where(lv == kv, 1, 0)
            rv = shf[pl.ds(15 + k, 16)]
            after = after + jnp.where(rv == kv, 1, 0)
        rb = plsc.load_gather(hist, [kv])
        rankb[pl.ds(off, 16)] = rb + within
        plsc.store_scatter(hist, [kv], rb + within + 1, mask=after == 0)
        return c

    lax.fori_loop(0, CHUNK // 16, body, 0)

    pltpu.sync_copy(hist, h_hbm.at[pl.ds(w * NB, NB)])
    pltpu.sync_copy(rankb, rank_hbm.at[pl.ds(w * CHUNK, CHUNK)])


@functools.partial(
    pl.kernel,
    mesh=_MESH,
    compiler_params=pltpu.CompilerParams(needs_layout_passes=False),
    out_type=jax.ShapeDtypeStruct((NE,), jnp.int32),
    scratch_types=[
        pltpu.VMEM((NB,), jnp.int32),
        pltpu.VMEM((CHUNK,), jnp.int32),
        pltpu.VMEM((CHUNK,), jnp.int32),
        pltpu.VMEM((CHUNK,), jnp.int32),
    ],
)
def _k2a(t_hbm, keys_hbm, rank_hbm, pos_hbm, tb, kb, rb, pb):
    w = _wid()
    pltpu.sync_copy(t_hbm.at[pl.ds(w * NB, NB)], tb)
    pltpu.sync_copy(keys_hbm.at[pl.ds(w * CHUNK, CHUNK)], kb)
    pltpu.sync_copy(rank_hbm.at[pl.ds(w * CHUNK, CHUNK)], rb)

    def body(i, c):
        off = i * 16
        kv = kb[pl.ds(off, 16)]
        tv = plsc.load_gather(tb, [kv])
        pb[pl.ds(off, 16)] = tv + rb[pl.ds(off, 16)]
        return c

    lax.fori_loop(0, CHUNK // 16, body, 0)
    pltpu.sync_copy(pb, pos_hbm.at[pl.ds(w * CHUNK, CHUNK)])


_NSUB = CHUNK // 128


@functools.partial(
    pl.kernel,
    mesh=_MESH,
    compiler_params=pltpu.CompilerParams(needs_layout_passes=False, use_tc_tiling_on_sc=False),
    out_type=jax.ShapeDtypeStruct((NE, 16), jnp.float32),
    scratch_types=[
        pltpu.VMEM((CHUNK,), jnp.int32),
        pltpu.VMEM((4 * CHUNK,), jnp.int32),
        pltpu.VMEM((_NSUB, 128), jnp.int32),
        pltpu.VMEM((128, 16), jnp.float32),
        pltpu.VMEM((512, 16), jnp.float32),
        pltpu.VMEM((128, 16), jnp.float32),
        pltpu.VMEM((32,), jnp.float32),
        pltpu.SemaphoreType.DMA,
        pltpu.SemaphoreType.DMA,
    ],
)
def _k2b(pcat_hbm, sel_hbm, nei_hbm, pos_hbm, out_hbm,
         selb, neib, posb, fb, nb, ob, shf, sem_g, sem_s):
    w = _wid()
    d_off = (w // 8) * N_NODES
    pltpu.sync_copy(sel_hbm.at[pl.ds(w * CHUNK, CHUNK)], selb)
    pltpu.sync_copy(nei_hbm.at[pl.ds(w * 4 * CHUNK, 4 * CHUNK)], neib)
    pltpu.sync_copy(pos_hbm.at[w], posb)

    def addsel(j, c):
        off = j * 16
        selb[pl.ds(off, 16)] = selb[pl.ds(off, 16)] + d_off
        return c

    lax.fori_loop(0, CHUNK // 16, addsel, 0)

    def addnei(j, c):
        off = j * 16
        neib[pl.ds(off, 16)] = neib[pl.ds(off, 16)] + d_off
        return c

    lax.fori_loop(0, 4 * CHUNK // 16, addnei, 0)

    shf[pl.ds(16, 16)] = jnp.zeros((16,), jnp.float32)

    def sub(s, c):
        cg = pltpu.async_copy(pcat_hbm.at[selb.at[pl.ds(s * 128, 128)]], fb, sem_g)
        cn0 = pltpu.async_copy(pcat_hbm.at[neib.at[pl.ds(s * 512, 128)]],
                               nb.at[pl.ds(0, 128)], sem_g)
        cn1 = pltpu.async_copy(pcat_hbm.at[neib.at[pl.ds(s * 512 + 128, 128)]],
                               nb.at[pl.ds(128, 128)], sem_g)
        cn2 = pltpu.async_copy(pcat_hbm.at[neib.at[pl.ds(s * 512 + 256, 128)]],
                               nb.at[pl.ds(256, 128)], sem_g)
        cn3 = pltpu.async_copy(pcat_hbm.at[neib.at[pl.ds(s * 512 + 384, 128)]],
                               nb.at[pl.ds(384, 128)], sem_g)
        cg.wait()
        cn0.wait()
        cn1.wait()
        cn2.wait()
        cn3.wait()

        def ebody(e, c2):
            base = e * 4
            acc = (nb[base] + nb[base + 1]) + (nb[base + 2] + nb[base + 3])
            shf[pl.ds(0, 16)] = acc
            sh = shf[pl.ds(8, 16)]
            ob[e] = fb[e] + sh
            return c2

        lax.fori_loop(0, 128, ebody, 0)

        cs = pltpu.async_copy(ob, out_hbm.at[posb.at[s]], sem_s)
        cs.wait()
        return c

    lax.fori_loop(0, _NSUB, sub, 0)


def kernel(is_last_layer, x, edge_index, edge_attr, p,
           p_focal_deg1, p_focal_deg2, p_focal_deg3, p_focal_deg4,
           nei_p_deg1, nei_p_deg2, nei_p_deg3, nei_p_deg4,
           nei_edge_attr_deg1, nei_edge_attr_deg2, nei_edge_attr_deg3, nei_edge_attr_deg4,
           selected_index_deg1, selected_index_deg2, selected_index_deg3, selected_index_deg4,
           nei_index_deg1, nei_index_deg2, nei_index_deg3, nei_index_deg4,
           save_score, W1, W2, W3, W4):
    sels = [selected_index_deg1, selected_index_deg2,
            selected_index_deg3, selected_index_deg4]
    neis = [nei_index_deg1, nei_index_deg2, nei_index_deg3, nei_index_deg4]

    wstack = jnp.stack([
        w.astype(jnp.float32).reshape(2, D_FEAT, NK).transpose(1, 0, 2).reshape(D_FEAT, 16)
        for w in (W1, W2, W3, W4)
    ])
    pcat = _project(x.astype(jnp.float32), wstack)

    pad_k = jnp.full((PAD_BLK - N_FOCAL,), PAD_KEY, jnp.int32)
    keys = jnp.concatenate(
        [jnp.concatenate([s.astype(jnp.int32), pad_k]) for s in sels])

    pad_z = jnp.zeros((PAD_BLK - N_FOCAL,), jnp.int32)
    sel_raw = jnp.concatenate(
        [jnp.concatenate([s.astype(jnp.int32), pad_z]) for s in sels])

    nei_blocks = []
    col = jnp.arange(4, dtype=jnp.int32)[None, :]
    row = jnp.arange(PAD_BLK, dtype=jnp.int32)[:, None]
    for d in range(1, 5):
        gflat = ((d - 1) * PAD_BLK + row) * 4 + col
        fill = 4 * N_NODES + gflat % ZROWS - (d - 1) * N_NODES
        real = neis[d - 1].astype(jnp.int32).reshape(N_FOCAL, d)
        real_p = jnp.pad(real, ((0, PAD_BLK - N_FOCAL), (0, 4 - d)))
        mask = (col < d) & (row < N_FOCAL)
        nei_blocks.append(jnp.where(mask, real_p, fill))
    nei_flat = jnp.concatenate(nei_blocks).reshape(4 * NE)

    hflat, rank = _k1(keys)
    tgrid = _offsets(hflat.reshape(NW, NB))
    pos = _k2a(tgrid.reshape(NW * NB), keys, rank)
    out_pad = _k2b(pcat, sel_raw, nei_flat, pos.reshape(NW, _NSUB, 128))
    return out_pad[:N_NODES, :NK]
